# baseline (speedup 1.0000x reference)
import functools

import jax
import jax.numpy as jnp
from jax import lax
from jax.experimental import pallas as pl
from jax.experimental.pallas import tpu as pltpu
from jax.experimental.pallas import tpu_sc as plsc

N = 10000
E = 320000
D = 128

NC = 2
NS = 16
NW = NC * NS
EPT = E // NW
C = 40
NCHUNK = EPT // C
NBUF = 6
NGROUP = NCHUNK // NBUF
NTAIL = NCHUNK - NGROUP * NBUF
CD = 125
NCHUNK_D = EPT // CD
ROWS_PT = N // NS


def _sc_mesh():
    return plsc.VectorSubcoreMesh(
        core_axis_name="c", subcore_axis_name="s", num_cores=NC, num_subcores=NS
    )


def _deg_call(src2d, dst2d, zeros16):
    @functools.partial(
        pl.kernel,
        out_type=jax.ShapeDtypeStruct((NC, 2, N, 16), jnp.float32),
        mesh=_sc_mesh(),
        compiler_params=pltpu.CompilerParams(use_tc_tiling_on_sc=False),
        scratch_types=[
            pltpu.VMEM_SHARED((N, 16), jnp.float32),
            pltpu.VMEM_SHARED((N, 16), jnp.float32),
            pltpu.VMEM((NCHUNK_D, CD), jnp.int32),
            pltpu.VMEM((NCHUNK_D, CD), jnp.int32),
            pltpu.VMEM((CD, 16), jnp.float32),
            pltpu.SemaphoreType.DMA,
            pltpu.SemaphoreType.DMA,
        ],
    )
    def k(src_hbm, dst_hbm, zero_hbm, out_hbm, hsrc, hdst, srcv, dstv, ones_v,
          sem_s, sem_d):
        cid = lax.axis_index("c")
        sid = lax.axis_index("s")
        row0 = sid * ROWS_PT

        pltpu.sync_copy(zero_hbm.at[pl.ds(row0, ROWS_PT)], hsrc.at[pl.ds(row0, ROWS_PT)])
        pltpu.sync_copy(zero_hbm.at[pl.ds(row0, ROWS_PT)], hdst.at[pl.ds(row0, ROWS_PT)])

        for i in range(CD):
            ones_v[i, :] = jnp.ones((16,), jnp.float32)

        erow0 = (cid * NS + sid) * NCHUNK_D
        pltpu.sync_copy(src_hbm.at[pl.ds(erow0, NCHUNK_D)], srcv)
        pltpu.sync_copy(dst_hbm.at[pl.ds(erow0, NCHUNK_D)], dstv)

        plsc.subcore_barrier()

        W = 8

        def body(g, carry):
            pltpu.async_copy(ones_v, hsrc.at[srcv.at[g]], sem_s, add=True)
            pltpu.async_copy(ones_v, hdst.at[dstv.at[g]], sem_d, add=True)

            @pl.when(g >= W)
            def _():
                pltpu.make_async_copy(ones_v, hsrc.at[srcv.at[g - W]], sem_s).wait()
                pltpu.make_async_copy(ones_v, hdst.at[dstv.at[g - W]], sem_d).wait()

            return carry

        lax.fori_loop(0, NCHUNK_D, body, 0)

        def drain(g, carry):
            pltpu.make_async_copy(ones_v, hsrc.at[srcv.at[g]], sem_s).wait()
            pltpu.make_async_copy(ones_v, hdst.at[dstv.at[g]], sem_d).wait()
            return carry

        lax.fori_loop(NCHUNK_D - W, NCHUNK_D, drain, 0)

        plsc.subcore_barrier()

        pltpu.sync_copy(hsrc.at[pl.ds(row0, ROWS_PT)], out_hbm.at[cid, 0, pl.ds(row0, ROWS_PT)])
        pltpu.sync_copy(hdst.at[pl.ds(row0, ROWS_PT)], out_hbm.at[cid, 1, pl.ds(row0, ROWS_PT)])

    return k(src2d, dst2d, zeros16)


def _agg_call(hs, src2d, dst2d, zeros128):
    @functools.partial(
        pl.kernel,
        out_type=jax.ShapeDtypeStruct((NC, N, D), jnp.float32),
        mesh=_sc_mesh(),
        compiler_params=pltpu.CompilerParams(use_tc_tiling_on_sc=False),
        scratch_types=[
            pltpu.VMEM_SHARED((N, D), jnp.float32),
            pltpu.VMEM((NCHUNK, C), jnp.int32),
            pltpu.VMEM((NCHUNK, C), jnp.int32),
            [pltpu.VMEM((C, D), jnp.float32) for _ in range(NBUF)],
            pltpu.SemaphoreType.DMA((NBUF,)),
            pltpu.SemaphoreType.DMA((NBUF,)),
        ],
    )
    def k(hs_hbm, src_hbm, dst_hbm, zero_hbm, out_hbm, acc, srcv, dstv, rows,
          gsem, ssem):
        cid = lax.axis_index("c")
        sid = lax.axis_index("s")
        row0 = sid * ROWS_PT

        pltpu.sync_copy(zero_hbm.at[pl.ds(row0, ROWS_PT)], acc.at[pl.ds(row0, ROWS_PT)])

        erow0 = (cid * NS + sid) * NCHUNK
        pltpu.sync_copy(src_hbm.at[pl.ds(erow0, NCHUNK)], srcv)
        pltpu.sync_copy(dst_hbm.at[pl.ds(erow0, NCHUNK)], dstv)

        plsc.subcore_barrier()

        def gather(g, b):
            pltpu.async_copy(hs_hbm.at[srcv.at[g]], rows[b], gsem.at[b])

        def gather_wait(g, b):
            pltpu.make_async_copy(hs_hbm.at[srcv.at[g]], rows[b], gsem.at[b]).wait()

        def scatter(g, b):
            pltpu.async_copy(rows[b], acc.at[dstv.at[g]], ssem.at[b], add=True)

        def scatter_wait(g, b):
            pltpu.make_async_copy(rows[b], acc.at[dstv.at[g]], ssem.at[b]).wait()

        for b in range(NBUF):
            gather(jnp.int32(b), b)

        def body(kk, carry):
            g0 = kk * NBUF
            for b in range(NBUF):
                gather_wait(g0 + b, b)
                scatter(g0 + b, b)
            for b in range(NBUF):
                scatter_wait(g0 + b, b)
                gn = g0 + NBUF + b

                @pl.when(gn < NCHUNK)
                def _():
                    gather(gn, b)
            return carry

        lax.fori_loop(0, NGROUP, body, 0)

        gl = jnp.int32(NGROUP * NBUF)
        for b in range(NTAIL):
            gather_wait(gl + b, b)
            scatter(gl + b, b)
        for b in range(NTAIL):
            scatter_wait(gl + b, b)

        plsc.subcore_barrier()

        pltpu.sync_copy(acc.at[pl.ds(row0, ROWS_PT)], out_hbm.at[cid, pl.ds(row0, ROWS_PT)])

    return k(hs, src2d, dst2d, zeros128)


_BR = 5000
_GRID = N // _BR


def _norm_from_deg(d16):
    n = jnp.where(d16 > 0.0, lax.rsqrt(jnp.maximum(d16, 1.0)), 0.0)
    return n[:, 0:1]


def _prep_body(x_ref, ds0_ref, ds1_ref, hs_ref):
    ns = _norm_from_deg(ds0_ref[...] + ds1_ref[...])
    hs_ref[...] = x_ref[...] * ns


def _prep_call(x, ds0, ds1):
    return pl.pallas_call(
        _prep_body,
        grid=(_GRID,),
        in_specs=[
            pl.BlockSpec((_BR, D), lambda i: (i, 0)),
            pl.BlockSpec((_BR, 16), lambda i: (i, 0)),
            pl.BlockSpec((_BR, 16), lambda i: (i, 0)),
        ],
        out_specs=pl.BlockSpec((_BR, D), lambda i: (i, 0)),
        out_shape=jax.ShapeDtypeStruct((N, D), jnp.float32),
    )(x, ds0, ds1)


def _layer_body(relu, want_hs, a0_ref, a1_ref, dd0_ref, dd1_ref, ds0_ref, ds1_ref,
                w_ref, b_ref, o_ref):
    nd = _norm_from_deg(dd0_ref[...] + dd1_ref[...])
    a = (a0_ref[...] + a1_ref[...]) * nd
    z = jnp.dot(a, w_ref[...], preferred_element_type=jnp.float32) + b_ref[...]
    if relu:
        z = jnp.maximum(z, 0.0)
    if want_hs:
        ns = _norm_from_deg(ds0_ref[...] + ds1_ref[...])
        z = z * ns
    o_ref[...] = z


def _layer_call(a0, a1, dd0, dd1, ds0, ds1, w, b, relu, want_hs):
    out_shape = [jax.ShapeDtypeStruct((N, D), jnp.float32)]
    out_specs = [pl.BlockSpec((_BR, D), lambda i: (i, 0))]
    return pl.pallas_call(
        functools.partial(_layer_body, relu, want_hs),
        grid=(_GRID,),
        in_specs=[
            pl.BlockSpec((_BR, D), lambda i: (i, 0)),
            pl.BlockSpec((_BR, D), lambda i: (i, 0)),
            pl.BlockSpec((_BR, 16), lambda i: (i, 0)),
            pl.BlockSpec((_BR, 16), lambda i: (i, 0)),
            pl.BlockSpec((_BR, 16), lambda i: (i, 0)),
            pl.BlockSpec((_BR, 16), lambda i: (i, 0)),
            pl.BlockSpec((D, D), lambda i: (0, 0)),
            pl.BlockSpec((1, D), lambda i: (0, 0)),
        ],
        out_specs=out_specs,
        out_shape=out_shape,
    )(a0, a1, dd0, dd1, ds0, ds1, w, b)


@jax.jit
def kernel(x, edge_index, W1, b1, W2, b2, W3, b3):
    src2d = edge_index[0].reshape(E // C, C)
    dst2d = edge_index[1].reshape(E // C, C)
    src2d_deg = edge_index[0].reshape(E // CD, CD)
    dst2d_deg = edge_index[1].reshape(E // CD, CD)
    zeros16 = jnp.zeros((N, 16), jnp.float32)
    zeros128 = jnp.zeros((N, D), jnp.float32)

    degs = _deg_call(src2d_deg, dst2d_deg, zeros16)
    ds0, ds1 = degs[0, 0], degs[1, 0]
    dd0, dd1 = degs[0, 1], degs[1, 1]

    hs = _prep_call(x, ds0, ds1)

    b1r = b1.reshape(1, D)
    b2r = b2.reshape(1, D)
    b3r = b3.reshape(1, D)

    aggp = _agg_call(hs, src2d, dst2d, zeros128)
    (hs,) = _layer_call(aggp[0], aggp[1], dd0, dd1, ds0, ds1, W1, b1r, True, True)

    aggp = _agg_call(hs, src2d, dst2d, zeros128)
    (hs,) = _layer_call(aggp[0], aggp[1], dd0, dd1, ds0, ds1, W2, b2r, True, True)

    aggp = _agg_call(hs, src2d, dst2d, zeros128)
    (h,) = _layer_call(aggp[0], aggp[1], dd0, dd1, ds0, ds1, W3, b3r, False, False)
    return h


# speedup vs baseline: 1.0182x; 1.0182x over previous
import functools

import jax
import jax.numpy as jnp
from jax import lax
from jax.experimental import pallas as pl
from jax.experimental.pallas import tpu as pltpu
from jax.experimental.pallas import tpu_sc as plsc

N = 10000
E = 320000
D = 128

NC = 2
NS = 16
NW = NC * NS
EPT = E // NW
C = 40
NCHUNK = EPT // C
NBUF = 6
NGROUP = NCHUNK // NBUF
NTAIL = NCHUNK - NGROUP * NBUF
CD = 125
NCHUNK_D = EPT // CD
ROWS_PT = N // NS


def _sc_mesh():
    return plsc.VectorSubcoreMesh(
        core_axis_name="c", subcore_axis_name="s", num_cores=NC, num_subcores=NS
    )


def _deg_call(src2d, dst2d, zeros16):
    @functools.partial(
        pl.kernel,
        out_type=jax.ShapeDtypeStruct((NC, 2, N, 16), jnp.float32),
        mesh=_sc_mesh(),
        compiler_params=pltpu.CompilerParams(use_tc_tiling_on_sc=False),
        scratch_types=[
            pltpu.VMEM_SHARED((N, 16), jnp.float32),
            pltpu.VMEM_SHARED((N, 16), jnp.float32),
            pltpu.VMEM((NCHUNK_D, CD), jnp.int32),
            pltpu.VMEM((NCHUNK_D, CD), jnp.int32),
            pltpu.VMEM((CD, 16), jnp.float32),
            pltpu.SemaphoreType.DMA,
            pltpu.SemaphoreType.DMA,
        ],
    )
    def k(src_hbm, dst_hbm, zero_hbm, out_hbm, hsrc, hdst, srcv, dstv, ones_v,
          sem_s, sem_d):
        cid = lax.axis_index("c")
        sid = lax.axis_index("s")
        row0 = sid * ROWS_PT

        erow0 = (cid * NS + sid) * NCHUNK_D
        pltpu.async_copy(zero_hbm.at[pl.ds(row0, ROWS_PT)], hsrc.at[pl.ds(row0, ROWS_PT)], sem_s)
        pltpu.async_copy(zero_hbm.at[pl.ds(row0, ROWS_PT)], hdst.at[pl.ds(row0, ROWS_PT)], sem_d)
        pltpu.async_copy(src_hbm.at[pl.ds(erow0, NCHUNK_D)], srcv, sem_s)
        pltpu.async_copy(dst_hbm.at[pl.ds(erow0, NCHUNK_D)], dstv, sem_d)

        for i in range(CD):
            ones_v[i, :] = jnp.ones((16,), jnp.float32)

        pltpu.make_async_copy(zero_hbm.at[pl.ds(row0, ROWS_PT)], hsrc.at[pl.ds(row0, ROWS_PT)], sem_s).wait()
        pltpu.make_async_copy(zero_hbm.at[pl.ds(row0, ROWS_PT)], hdst.at[pl.ds(row0, ROWS_PT)], sem_d).wait()
        pltpu.make_async_copy(src_hbm.at[pl.ds(erow0, NCHUNK_D)], srcv, sem_s).wait()
        pltpu.make_async_copy(dst_hbm.at[pl.ds(erow0, NCHUNK_D)], dstv, sem_d).wait()

        plsc.subcore_barrier()

        W = 8

        def body(g, carry):
            pltpu.async_copy(ones_v, hsrc.at[srcv.at[g]], sem_s, add=True)
            pltpu.async_copy(ones_v, hdst.at[dstv.at[g]], sem_d, add=True)

            @pl.when(g >= W)
            def _():
                pltpu.make_async_copy(ones_v, hsrc.at[srcv.at[g - W]], sem_s).wait()
                pltpu.make_async_copy(ones_v, hdst.at[dstv.at[g - W]], sem_d).wait()

            return carry

        lax.fori_loop(0, NCHUNK_D, body, 0)

        def drain(g, carry):
            pltpu.make_async_copy(ones_v, hsrc.at[srcv.at[g]], sem_s).wait()
            pltpu.make_async_copy(ones_v, hdst.at[dstv.at[g]], sem_d).wait()
            return carry

        lax.fori_loop(NCHUNK_D - W, NCHUNK_D, drain, 0)

        plsc.subcore_barrier()

        pltpu.sync_copy(hsrc.at[pl.ds(row0, ROWS_PT)], out_hbm.at[cid, 0, pl.ds(row0, ROWS_PT)])
        pltpu.sync_copy(hdst.at[pl.ds(row0, ROWS_PT)], out_hbm.at[cid, 1, pl.ds(row0, ROWS_PT)])

    return k(src2d, dst2d, zeros16)


def _agg_call(hs, src2d, dst2d, zeros128):
    @functools.partial(
        pl.kernel,
        out_type=jax.ShapeDtypeStruct((NC, N, D), jnp.float32),
        mesh=_sc_mesh(),
        compiler_params=pltpu.CompilerParams(use_tc_tiling_on_sc=False),
        scratch_types=[
            pltpu.VMEM_SHARED((N, D), jnp.float32),
            pltpu.VMEM((NCHUNK, C), jnp.int32),
            pltpu.VMEM((NCHUNK, C), jnp.int32),
            [pltpu.VMEM((C, D), jnp.float32) for _ in range(NBUF)],
            pltpu.SemaphoreType.DMA((NBUF,)),
            pltpu.SemaphoreType.DMA((NBUF,)),
        ],
    )
    def k(hs_hbm, src_hbm, dst_hbm, zero_hbm, out_hbm, acc, srcv, dstv, rows,
          gsem, ssem):
        cid = lax.axis_index("c")
        sid = lax.axis_index("s")
        row0 = sid * ROWS_PT

        erow0 = (cid * NS + sid) * NCHUNK
        pltpu.async_copy(zero_hbm.at[pl.ds(row0, ROWS_PT)], acc.at[pl.ds(row0, ROWS_PT)], gsem.at[0])
        pltpu.async_copy(src_hbm.at[pl.ds(erow0, NCHUNK)], srcv, gsem.at[1])
        pltpu.async_copy(dst_hbm.at[pl.ds(erow0, NCHUNK)], dstv, gsem.at[2])
        pltpu.make_async_copy(zero_hbm.at[pl.ds(row0, ROWS_PT)], acc.at[pl.ds(row0, ROWS_PT)], gsem.at[0]).wait()
        pltpu.make_async_copy(src_hbm.at[pl.ds(erow0, NCHUNK)], srcv, gsem.at[1]).wait()
        pltpu.make_async_copy(dst_hbm.at[pl.ds(erow0, NCHUNK)], dstv, gsem.at[2]).wait()

        plsc.subcore_barrier()

        def gather(g, b):
            pltpu.async_copy(hs_hbm.at[srcv.at[g]], rows[b], gsem.at[b])

        def gather_wait(g, b):
            pltpu.make_async_copy(hs_hbm.at[srcv.at[g]], rows[b], gsem.at[b]).wait()

        def scatter(g, b):
            pltpu.async_copy(rows[b], acc.at[dstv.at[g]], ssem.at[b], add=True)

        def scatter_wait(g, b):
            pltpu.make_async_copy(rows[b], acc.at[dstv.at[g]], ssem.at[b]).wait()

        for b in range(NBUF):
            gather(jnp.int32(b), b)

        def body(kk, carry):
            g0 = kk * NBUF
            for b in range(NBUF):
                gather_wait(g0 + b, b)
                scatter(g0 + b, b)
            for b in range(NBUF):
                scatter_wait(g0 + b, b)
                gn = g0 + NBUF + b

                @pl.when(gn < NCHUNK)
                def _():
                    gather(gn, b)
            return carry

        lax.fori_loop(0, NGROUP, body, 0)

        gl = jnp.int32(NGROUP * NBUF)
        for b in range(NTAIL):
            gather_wait(gl + b, b)
            scatter(gl + b, b)
        for b in range(NTAIL):
            scatter_wait(gl + b, b)

        plsc.subcore_barrier()

        pltpu.sync_copy(acc.at[pl.ds(row0, ROWS_PT)], out_hbm.at[cid, pl.ds(row0, ROWS_PT)])

    return k(hs, src2d, dst2d, zeros128)


_BR = 2000
_GRID = N // _BR


def _norm_from_deg(d16):
    n = jnp.where(d16 > 0.0, lax.rsqrt(jnp.maximum(d16, 1.0)), 0.0)
    return n[:, 0:1]


def _prep_body(x_ref, ds0_ref, ds1_ref, hs_ref):
    ns = _norm_from_deg(ds0_ref[...] + ds1_ref[...])
    hs_ref[...] = x_ref[...] * ns


def _prep_call(x, ds0, ds1):
    return pl.pallas_call(
        _prep_body,
        grid=(_GRID,),
        in_specs=[
            pl.BlockSpec((_BR, D), lambda i: (i, 0)),
            pl.BlockSpec((_BR, 16), lambda i: (i, 0)),
            pl.BlockSpec((_BR, 16), lambda i: (i, 0)),
        ],
        out_specs=pl.BlockSpec((_BR, D), lambda i: (i, 0)),
        out_shape=jax.ShapeDtypeStruct((N, D), jnp.float32),
    )(x, ds0, ds1)


def _layer_body(relu, want_hs, a0_ref, a1_ref, dd0_ref, dd1_ref, ds0_ref, ds1_ref,
                w_ref, b_ref, o_ref):
    nd = _norm_from_deg(dd0_ref[...] + dd1_ref[...])
    a = (a0_ref[...] + a1_ref[...]) * nd
    z = jnp.dot(a, w_ref[...], preferred_element_type=jnp.float32) + b_ref[...]
    if relu:
        z = jnp.maximum(z, 0.0)
    if want_hs:
        ns = _norm_from_deg(ds0_ref[...] + ds1_ref[...])
        z = z * ns
    o_ref[...] = z


def _layer_call(a0, a1, dd0, dd1, ds0, ds1, w, b, relu, want_hs):
    out_shape = [jax.ShapeDtypeStruct((N, D), jnp.float32)]
    out_specs = [pl.BlockSpec((_BR, D), lambda i: (i, 0))]
    return pl.pallas_call(
        functools.partial(_layer_body, relu, want_hs),
        grid=(_GRID,),
        in_specs=[
            pl.BlockSpec((_BR, D), lambda i: (i, 0)),
            pl.BlockSpec((_BR, D), lambda i: (i, 0)),
            pl.BlockSpec((_BR, 16), lambda i: (i, 0)),
            pl.BlockSpec((_BR, 16), lambda i: (i, 0)),
            pl.BlockSpec((_BR, 16), lambda i: (i, 0)),
            pl.BlockSpec((_BR, 16), lambda i: (i, 0)),
            pl.BlockSpec((D, D), lambda i: (0, 0)),
            pl.BlockSpec((1, D), lambda i: (0, 0)),
        ],
        out_specs=out_specs,
        out_shape=out_shape,
    )(a0, a1, dd0, dd1, ds0, ds1, w, b)


@jax.jit
def kernel(x, edge_index, W1, b1, W2, b2, W3, b3):
    src2d = edge_index[0].reshape(E // C, C)
    dst2d = edge_index[1].reshape(E // C, C)
    src2d_deg = edge_index[0].reshape(E // CD, CD)
    dst2d_deg = edge_index[1].reshape(E // CD, CD)
    zeros16 = jnp.zeros((N, 16), jnp.float32)
    zeros128 = jnp.zeros((N, D), jnp.float32)

    degs = _deg_call(src2d_deg, dst2d_deg, zeros16)
    ds0, ds1 = degs[0, 0], degs[1, 0]
    dd0, dd1 = degs[0, 1], degs[1, 1]

    hs = _prep_call(x, ds0, ds1)

    b1r = b1.reshape(1, D)
    b2r = b2.reshape(1, D)
    b3r = b3.reshape(1, D)

    aggp = _agg_call(hs, src2d, dst2d, zeros128)
    (hs,) = _layer_call(aggp[0], aggp[1], dd0, dd1, ds0, ds1, W1, b1r, True, True)

    aggp = _agg_call(hs, src2d, dst2d, zeros128)
    (hs,) = _layer_call(aggp[0], aggp[1], dd0, dd1, ds0, ds1, W2, b2r, True, True)

    aggp = _agg_call(hs, src2d, dst2d, zeros128)
    (h,) = _layer_call(aggp[0], aggp[1], dd0, dd1, ds0, ds1, W3, b3r, False, False)
    return h
